# baseline (speedup 1.0000x reference)
import functools

import jax
import jax.numpy as jnp
from jax import lax
from jax.experimental import pallas as pl
from jax.experimental.pallas import tpu as pltpu
from jax.experimental.pallas import tpu_sc as plsc

VOCAB_ROWS = 32
EMBED_DIM = 64
BATCH = 4096
SEQ = 200
TOTAL = BATCH * SEQ

_info = plsc.get_sparse_core_info()
_NC = _info.num_cores
_NS = _info.num_subcores
_NW = _NC * _NS
_L = _info.num_lanes
PER_W = TOTAL // _NW
BUF_ROWS = 128
NSTEP = PER_W // BUF_ROWS
NGROUP = BUF_ROWS // _L
NBUF = 4


def _make_kernel():
    mesh = plsc.VectorSubcoreMesh(core_axis_name="c", subcore_axis_name="s")

    @functools.partial(
        pl.kernel,
        mesh=mesh,
        out_type=jax.ShapeDtypeStruct((TOTAL * EMBED_DIM,), jnp.float32),
        compiler_params=pltpu.CompilerParams(
            use_tc_tiling_on_sc=False, needs_layout_passes=False),
        scratch_types=[
            pltpu.VMEM((PER_W,), jnp.int32),
            pltpu.VMEM((NBUF, BUF_ROWS * EMBED_DIM), jnp.float32),
            pltpu.VMEM((VOCAB_ROWS * EMBED_DIM,), jnp.float32),
        ]
        + [pltpu.SemaphoreType.DMA] * NBUF,
    )
    def k(idx_hbm, table_hbm, out_hbm, idx_v, rows, table_v,
          o0, o1, o2, o3):
        osem = [o0, o1, o2, o3]
        wid = lax.axis_index("s") * _NC + lax.axis_index("c")
        base = wid * PER_W
        obase = base * EMBED_DIM

        pltpu.sync_copy(table_hbm, table_v)
        pltpu.sync_copy(idx_hbm.at[pl.ds(base, PER_W)], idx_v)

        lanes = lax.iota(jnp.int32, _L)
        ovec = lanes * EMBED_DIM

        def compute(s, b):
            buf = rows.at[b]

            def group(g, carry):
                idx_vec = idx_v[pl.ds(s * BUF_ROWS + g * _L, _L)]
                gvec = idx_vec << 6
                dvec = ovec + g * (_L * EMBED_DIM)
                for c in range(EMBED_DIM):
                    cvec = (lanes + c) & (EMBED_DIM - 1)
                    v = plsc.load_gather(table_v, [gvec + cvec])
                    plsc.store_scatter(buf, [dvec + cvec], v)
                return carry

            lax.fori_loop(0, NGROUP, group, 0)

        def write(s, b, start):
            cp = pltpu.make_async_copy(
                rows.at[b],
                out_hbm.at[pl.ds(obase + s * (BUF_ROWS * EMBED_DIM),
                                 BUF_ROWS * EMBED_DIM)],
                osem[b],
            )
            cp.start() if start else cp.wait()

        for b in range(NBUF):
            compute(b, b)
            write(b, b, True)

        def body(i, carry):
            sbase = i * NBUF
            for b in range(NBUF):
                s = sbase + b
                write(s - NBUF, b, False)
                compute(s, b)
                write(s, b, True)
            return carry

        lax.fori_loop(1, NSTEP // NBUF, body, 0)

        for b in range(NBUF):
            write(NSTEP - NBUF + b, b, False)

    return k


_sc_gather = _make_kernel()


def kernel(inputs, embedding_table):
    idx = inputs.reshape(TOTAL)
    table = embedding_table.reshape(VOCAB_ROWS * EMBED_DIM)
    out = _sc_gather(idx, table)
    return out.reshape(BATCH, SEQ, EMBED_DIM)


# speedup vs baseline: 1.6040x; 1.6040x over previous
import functools

import jax
import jax.numpy as jnp
from jax import lax
from jax.experimental import pallas as pl
from jax.experimental.pallas import tpu as pltpu
from jax.experimental.pallas import tpu_sc as plsc

VOCAB_ROWS = 32
EMBED_DIM = 64
BATCH = 4096
SEQ = 200
TOTAL = BATCH * SEQ

_info = plsc.get_sparse_core_info()
_NC = _info.num_cores
_NS = _info.num_subcores
_NW = _NC * _NS
_L = _info.num_lanes
PER_W = TOTAL // _NW
CHUNK = 128
NSTEP = PER_W // CHUNK
NGROUP = CHUNK // _L
SPS = 3
CPS = 2
NSUP = NSTEP // (SPS + CPS)
S_CHUNKS = SPS * NSUP
C_BASE = S_CHUNKS


def _make_kernel():
    mesh = plsc.VectorSubcoreMesh(core_axis_name="c", subcore_axis_name="s")

    @functools.partial(
        pl.kernel,
        mesh=mesh,
        out_type=jax.ShapeDtypeStruct((TOTAL, EMBED_DIM), jnp.float32),
        compiler_params=pltpu.CompilerParams(
            use_tc_tiling_on_sc=False, needs_layout_passes=False),
        scratch_types=[
            pltpu.VMEM((PER_W,), jnp.int32),
            pltpu.VMEM((2, SPS, CHUNK, EMBED_DIM), jnp.float32),
            pltpu.VMEM((CPS, CHUNK, EMBED_DIM), jnp.float32),
            pltpu.VMEM((VOCAB_ROWS, EMBED_DIM), jnp.float32),
            pltpu.VMEM_SHARED((VOCAB_ROWS, EMBED_DIM), jnp.float32),
        ]
        + [pltpu.SemaphoreType.DMA] * 6,
    )
    def k(idx_hbm, table_hbm, out_hbm, idx_f, s_rows, c_rows,
          table_v, table_sh, ga, gb, oa, ob, oc0, oc1):
        gsem = [ga, gb]
        osem = [oa, ob]
        ocsem = [oc0, oc1]
        sid = lax.axis_index("s")
        wid = sid * _NC + lax.axis_index("c")
        base = wid * PER_W

        @pl.when(sid == 0)
        def _():
            pltpu.sync_copy(table_hbm, table_sh)

        pltpu.sync_copy(table_hbm, table_v)
        pltpu.sync_copy(idx_hbm.at[pl.ds(base, PER_W)], idx_f)
        plsc.subcore_barrier()

        lanes = lax.iota(jnp.int32, _L)

        def sg(q, t, j, start):
            cp = pltpu.make_async_copy(
                table_sh.at[idx_f.at[pl.ds(q * CHUNK, CHUNK)]],
                s_rows.at[t, j], gsem[t])
            cp.start() if start else cp.wait()

        def sw(q, t, j, start):
            cp = pltpu.make_async_copy(
                s_rows.at[t, j],
                out_hbm.at[pl.ds(base + q * CHUNK, CHUNK)], osem[t])
            cp.start() if start else cp.wait()

        def comp(c, u):
            buf = c_rows.at[u]

            @plsc.parallel_loop(0, NGROUP, unroll=1)
            def group(g):
                idx_vec = idx_f[pl.ds(c * CHUNK + g * _L, _L)]
                rowv = g * _L + lanes
                for cc in range(EMBED_DIM):
                    cvec = (lanes + cc) & (EMBED_DIM - 1)
                    v = plsc.load_gather(table_v, [idx_vec, cvec])
                    plsc.store_scatter(buf, [rowv, cvec], v)

        def cw(c, u, start):
            cp = pltpu.make_async_copy(
                c_rows.at[u],
                out_hbm.at[pl.ds(base + c * CHUNK, CHUNK)], ocsem[u])
            cp.start() if start else cp.wait()

        for j in range(SPS):
            sg(j, 0, j, True)
        for u in range(CPS):
            comp(C_BASE + u, u)
            cw(C_BASE + u, u, True)

        for j in range(SPS):
            sg(SPS + j, 1, j, True)
        for j in range(SPS):
            sg(j, 0, j, False)
            sw(j, 0, j, True)
        for u in range(CPS):
            cw(C_BASE + u, u, False)
            comp(C_BASE + CPS + u, u)
            cw(C_BASE + CPS + u, u, True)

        def body(kk, carry):
            for t in range(2):
                i = 2 * kk + t
                for j in range(SPS):
                    sw(SPS * (i - 2) + j, t, j, False)
                for j in range(SPS):
                    sg(SPS * i + j, t, j, True)
                for j in range(SPS):
                    sg(SPS * (i - 1) + j, 1 - t, j, False)
                for j in range(SPS):
                    sw(SPS * (i - 1) + j, 1 - t, j, True)
                for u in range(CPS):
                    cw(C_BASE + CPS * (i - 1) + u, u, False)
                    comp(C_BASE + CPS * i + u, u)
                    cw(C_BASE + CPS * i + u, u, True)
            return carry

        lax.fori_loop(1, NSUP // 2, body, 0)

        last1 = SPS * (NSUP - 1)
        for j in range(SPS):
            sg(last1 + j, 1, j, False)
            sw(last1 + j, 1, j, True)
        for j in range(SPS):
            sw(SPS * (NSUP - 2) + j, 0, j, False)
        for j in range(SPS):
            sw(last1 + j, 1, j, False)
        for u in range(CPS):
            cw(C_BASE + CPS * (NSUP - 1) + u, u, False)

    return k


_sc_gather = _make_kernel()


def kernel(inputs, embedding_table):
    idx = inputs.reshape(TOTAL)
    out = _sc_gather(idx, embedding_table)
    return out.reshape(BATCH, SEQ, EMBED_DIM)


# speedup vs baseline: 1.6546x; 1.0316x over previous
import functools

import jax
import jax.numpy as jnp
from jax import lax
from jax.experimental import pallas as pl
from jax.experimental.pallas import tpu as pltpu
from jax.experimental.pallas import tpu_sc as plsc

VOCAB_ROWS = 32
EMBED_DIM = 64
BATCH = 4096
SEQ = 200
TOTAL = BATCH * SEQ

_info = plsc.get_sparse_core_info()
_NC = _info.num_cores
_NS = _info.num_subcores
_L = _info.num_lanes
CHUNK = 128
N_CHUNKS = TOTAL // CHUNK
NGROUP = CHUNK // _L
HALF = _NS // 2
NTILE = HALF * _NC
SPT = 304
CPT = (N_CHUNKS - NTILE * SPT) // NTILE
C_BASE = NTILE * SPT
NBUF = 4
SKEW = 2


def _make_kernel():
    mesh = plsc.VectorSubcoreMesh(core_axis_name="c", subcore_axis_name="s")

    @functools.partial(
        pl.kernel,
        mesh=mesh,
        out_type=jax.ShapeDtypeStruct((TOTAL, EMBED_DIM), jnp.float32),
        compiler_params=pltpu.CompilerParams(
            use_tc_tiling_on_sc=False, needs_layout_passes=False),
        scratch_types=[
            pltpu.VMEM((SPT * CHUNK,), jnp.int32),
            pltpu.VMEM((NBUF, CHUNK, EMBED_DIM), jnp.float32),
            pltpu.VMEM((2, CHUNK, EMBED_DIM), jnp.float32),
            pltpu.VMEM((VOCAB_ROWS, EMBED_DIM), jnp.float32),
            pltpu.VMEM_SHARED((VOCAB_ROWS, EMBED_DIM), jnp.float32),
        ]
        + [pltpu.SemaphoreType.DMA] * 10,
    )
    def k(idx_hbm, table_hbm, out_hbm, idx_v, s_rows, c_rows, table_v,
          table_sh, g0, g1, g2, g3, o0, o1, o2, o3, oc0, oc1):
        gsem = [g0, g1, g2, g3]
        osem = [o0, o1, o2, o3]
        ocsem = [oc0, oc1]
        sid = lax.axis_index("s")
        cid = lax.axis_index("c")

        @pl.when(sid == 0)
        def _():
            pltpu.sync_copy(table_hbm, table_sh)

        plsc.subcore_barrier()

        lanes = lax.iota(jnp.int32, _L)

        @pl.when(sid < HALF)
        def _stream_role():
            stid = sid * _NC + cid
            first = stid * SPT
            pltpu.sync_copy(idx_hbm.at[pl.ds(first * CHUNK, SPT * CHUNK)],
                            idx_v)

            def sg(q, b, start):
                cp = pltpu.make_async_copy(
                    table_sh.at[idx_v.at[pl.ds(q * CHUNK, CHUNK)]],
                    s_rows.at[b], gsem[b])
                cp.start() if start else cp.wait()

            def sw(q, b, start):
                cp = pltpu.make_async_copy(
                    s_rows.at[b],
                    out_hbm.at[pl.ds((first + q) * CHUNK, CHUNK)], osem[b])
                cp.start() if start else cp.wait()

            for b in range(NBUF):
                sg(b, b, True)
            for b in range(SKEW):
                sg(b, b, False)
                sw(b, b, True)

            def body(i, carry):
                qb = i * NBUF
                for b in range(NBUF):
                    q = qb + b
                    sw(q - NBUF, b, False)
                    sg(q, b, True)
                    qw = q - SKEW
                    bw = (b + NBUF - SKEW) % NBUF
                    sg(qw, bw, False)
                    sw(qw, bw, True)
                return carry

            lax.fori_loop(1, SPT // NBUF, body, 0)

            lastq = SPT - NBUF
            for b in range(SKEW, NBUF):
                sg(lastq + b, b, False)
                sw(lastq + b, b, True)
            for b in range(NBUF):
                sw(lastq + b, b, False)

        @pl.when(sid >= HALF)
        def _vector_role():
            ctid = (sid - HALF) * _NC + cid
            first = C_BASE + ctid * CPT
            pltpu.sync_copy(table_hbm, table_v)
            pltpu.sync_copy(idx_hbm.at[pl.ds(first * CHUNK, CPT * CHUNK)],
                            idx_v.at[pl.ds(0, CPT * CHUNK)])

            def comp(c, u):
                buf = c_rows.at[u]

                @plsc.parallel_loop(0, NGROUP, unroll=1)
                def group(g):
                    idx_vec = idx_v[pl.ds(c * CHUNK + g * _L, _L)]
                    rowv = g * _L + lanes
                    for cc in range(EMBED_DIM):
                        cvec = (lanes + cc) & (EMBED_DIM - 1)
                        v = plsc.load_gather(table_v, [idx_vec, cvec])
                        plsc.store_scatter(buf, [rowv, cvec], v)

            def cw(c, u, start):
                cp = pltpu.make_async_copy(
                    c_rows.at[u],
                    out_hbm.at[pl.ds((first + c) * CHUNK, CHUNK)], ocsem[u])
                cp.start() if start else cp.wait()

            for u in range(2):
                comp(u, u)
                cw(u, u, True)

            def body(i, carry):
                cb = i * 2
                for u in range(2):
                    c = cb + u
                    cw(c - 2, u, False)
                    comp(c, u)
                    cw(c, u, True)
                return carry

            lax.fori_loop(1, CPT // 2, body, 0)

            for u in range(2):
                cw(CPT - 2 + u, u, False)

    return k


_sc_gather = _make_kernel()


def kernel(inputs, embedding_table):
    idx = inputs.reshape(TOTAL)
    out = _sc_gather(idx, embedding_table)
    return out.reshape(BATCH, SEQ, EMBED_DIM)
